# baseline (speedup 1.0000x reference)
import functools
import math

import jax
import jax.numpy as jnp
from jax import lax
from jax.experimental import pallas as pl
from jax.experimental.pallas import tpu as pltpu
from jax.experimental.pallas import tpu_sc as plsc

R = 3.0
LENGTH_ENC_DIM = 64
N_SIN = 10000
HALF_DIM = LENGTH_ENC_DIM // 2
EMB = math.log(N_SIN) / (HALF_DIM - 1)
SQ3 = math.sqrt(3.0)
SQ5 = math.sqrt(5.0)

SA, SB = 0.8 * R, 0.99 * R
NA, NB = 0.05 * R, 0.15 * R

NUM_CORES = 2
NUM_SUBCORES = 16
NW = NUM_CORES * NUM_SUBCORES

ROW_W = 16

TWO_PI_HI = 6.28125
TWO_PI_MID = 0.0019350051879882812
TWO_PI_LO = 3.0199159795074593e-07
INV_TWO_PI = 1.0 / (2.0 * math.pi)
SIN_C = (0.9999999528394287, -0.16666629708156558, 0.008332868412415909,
         -0.00019819996482129334, 2.711761652809948e-06,
         -2.082389059190004e-08)
COS_C = (0.9999999915888924, -0.49999991137407596, 0.0416665153020704,
         -0.0013887924190438086, 2.477237725272657e-05,
         -2.710259876049026e-07, 1.7328045475960078e-09)


def _poly_u(coeffs, u):
    r = coeffs[-1]
    for c in coeffs[-2::-1]:
        r = r * u + c
    return r


def _cos_0_pi(t):
    return _poly_u(COS_C, t * t)


def _sincos(ang):
    k = jnp.rint(ang * INV_TWO_PI)
    th = ang - k * TWO_PI_HI
    th = th - k * TWO_PI_MID
    th = th - k * TWO_PI_LO
    u = th * th
    return _poly_u(SIN_C, u) * th, _poly_u(COS_C, u)


def _sc_gather_make(N, E, chunk):
    assert E % NW == 0
    per_w = E // NW
    assert per_w % chunk == 0 and chunk % 8 == 0 and chunk % 4 == 0
    n_chunks = per_w // chunk
    mesh = plsc.VectorSubcoreMesh(core_axis_name="c", subcore_axis_name="s")

    psub = -(-N // NUM_SUBCORES)
    psub = -(-psub // 16) * 16
    nt = psub * NUM_SUBCORES

    @functools.partial(
        pl.kernel,
        out_type=[
            jax.ShapeDtypeStruct((3 * E,), jnp.float32),
            jax.ShapeDtypeStruct((NUM_CORES, nt, ROW_W), jnp.float32),
            jax.ShapeDtypeStruct((NUM_CORES, nt, ROW_W), jnp.float32),
        ],
        mesh=mesh,
        compiler_params=pltpu.CompilerParams(
            use_tc_tiling_on_sc=False, needs_layout_passes=False),
        scratch_types=[
            pltpu.VMEM((chunk,), jnp.int32),
            pltpu.VMEM((chunk,), jnp.int32),
            pltpu.VMEM((chunk, ROW_W), jnp.float32),
            pltpu.VMEM((chunk, ROW_W), jnp.float32),
            pltpu.VMEM((3 * chunk,), jnp.float32),
            pltpu.VMEM((3, psub), jnp.float32),
            pltpu.VMEM((psub, ROW_W), jnp.float32),
            pltpu.SemaphoreType.DMA,
            pltpu.SemaphoreType.DMA,
        ],
    )
    def sc_gather(xst_hbm, xdt_hbm, es_hbm, ed_hbm,
                  out_hbm, xs_tbl, xd_tbl,
                  idx_s, idx_d, rows_s, rows_d, diff, plane_v, tbl_v,
                  sem_s, sem_d):
        sid = lax.axis_index("s")
        core = lax.axis_index("c")
        wid = sid * NUM_CORES + core
        lane = lax.broadcasted_iota(jnp.int32, (16,), 0)

        n0 = sid * psub
        pcols = [lane - lane + p for p in range(3)]
        for xt, tbl in ((xst_hbm, xs_tbl), (xdt_hbm, xd_tbl)):
            for p in range(3):
                pltpu.sync_copy(xt.at[p, pl.ds(n0, psub)],
                                plane_v.at[p, pl.ds(0, psub)])

            def bbody(j, _):
                row = j * 16 + lane
                for p in range(3):
                    v = plane_v[p, pl.ds(j * 16, 16)]
                    plsc.store_scatter(tbl_v, [row, pcols[p]], v)
                return ()

            lax.fori_loop(0, psub // 16, bbody, (), unroll=4)
            pltpu.sync_copy(tbl_v, tbl.at[core, pl.ds(n0, psub)])
        plsc.subcore_barrier()
        xs_my = xs_tbl.at[core]
        xd_my = xd_tbl.at[core]

        for c in range(n_chunks):
            base = wid * per_w + c * chunk
            pltpu.sync_copy(es_hbm.at[pl.ds(base, chunk)], idx_s)
            pltpu.sync_copy(ed_hbm.at[pl.ds(base, chunk)], idx_d)
            cp_s = pltpu.async_copy(xs_my.at[idx_s], rows_s, sem_s)
            cp_d = pltpu.async_copy(xd_my.at[idx_d], rows_d, sem_d)
            cp_s.wait()
            cp_d.wait()

            def body(i, _):
                row = i * 16 + lane
                for p in range(3):
                    a = plsc.load_gather(rows_s, [row, pcols[p]])
                    b = plsc.load_gather(rows_d, [row, pcols[p]])
                    diff[pl.ds(p * chunk + i * 16, 16)] = a - b
                return ()

            n_full = chunk // 16
            lax.fori_loop(0, n_full, body, (), unroll=4)
            if chunk % 16:
                row = n_full * 16 + lane
                tmask = lane < (chunk % 16)
                for p in range(3):
                    a = plsc.load_gather(rows_s, [row, pcols[p]], mask=tmask)
                    b = plsc.load_gather(rows_d, [row, pcols[p]], mask=tmask)
                    plsc.store_scatter(
                        diff, [p * chunk + n_full * 16 + lane], a - b,
                        mask=tmask)
            for p in range(3):
                pltpu.sync_copy(
                    diff.at[pl.ds(p * chunk, chunk)],
                    out_hbm.at[pl.ds(p * E + base, chunk)])

    return sc_gather


def _tc_math_body(v_ref, sh_ref, len_ref, scal_ref, cs_ref, cn_ref):
    i = pl.program_id(0)
    x = v_ref[0:1, :]
    y = v_ref[1:2, :]
    z = v_ref[2:3, :]
    l2 = x * x + y * y + z * z
    length = jnp.sqrt(l2 + 1e-12)
    inv = 1.0 / length
    ux = x * inv
    uy = y * inv
    uz = z * inv

    tf = jnp.clip((length - SA) / (SB - SA), 0.0, 1.0)
    fall = 0.5 * (1.0 + _cos_0_pi(jnp.pi * tf))
    tr = jnp.clip((length - NA) / (NB - NA), 0.0, 1.0)
    rise = 0.5 * (1.0 - _cos_0_pi(jnp.pi * tr))
    cs = fall
    cn = rise * fall
    blk = length.shape[1]
    off = pl.multiple_of(i * blk, blk)
    cs_ref[pl.ds(off, blk)] = cs.reshape(blk)
    cn_ref[pl.ds(off, blk)] = cn.reshape(blk)
    len_ref[pl.ds(off, blk)] = length.reshape(blk)

    ii = lax.broadcasted_iota(jnp.int32, (HALF_DIM, 1), 0)
    freqs = jnp.exp(ii.astype(jnp.float32) * (-EMB))
    ang = (length * (N_SIN / R)) * freqs
    sin_a, cos_a = _sincos(ang)
    scal_ref[...] = jnp.concatenate([sin_a, cos_a], axis=0)

    sh1x = SQ3 * ux * cn
    sh1y = SQ3 * uy * cn
    sh1z = SQ3 * uz * cn
    c5 = SQ5 * cn
    sh2a = c5 * SQ3 * ux * uz
    sh2b = c5 * SQ3 * ux * uy
    sh2c = c5 * (uy * uy - 0.5 * (ux * ux + uz * uz))
    sh2d = c5 * SQ3 * uy * uz
    sh2e = c5 * (SQ3 / 2.0) * (uz * uz - ux * ux)
    sh_ref[...] = jnp.concatenate(
        [cs, sh1x, sh1y, sh1z, sh2a, sh2b, sh2c, sh2d, sh2e], axis=0)


def _tc_math(vec_planes, blk):
    E = vec_planes.shape[1]
    assert E % blk == 0 and blk % 128 == 0
    grid = (E // blk,)
    out_shapes = [
        jax.ShapeDtypeStruct((9, E), jnp.float32),
        jax.ShapeDtypeStruct((E,), jnp.float32),
        jax.ShapeDtypeStruct((LENGTH_ENC_DIM, E), jnp.float32),
        jax.ShapeDtypeStruct((E,), jnp.float32),
        jax.ShapeDtypeStruct((E,), jnp.float32),
    ]
    out_specs = [
        pl.BlockSpec((9, blk), lambda i: (0, i)),
        pl.BlockSpec((E, ), lambda i: (0,)),
        pl.BlockSpec((LENGTH_ENC_DIM, blk), lambda i: (0, i)),
        pl.BlockSpec((E,), lambda i: (0,)),
        pl.BlockSpec((E,), lambda i: (0,)),
    ]
    return pl.pallas_call(
        _tc_math_body,
        grid=grid,
        in_specs=[pl.BlockSpec((3, blk), lambda i: (0, i))],
        out_specs=out_specs,
        out_shape=out_shapes,
    )(vec_planes)


def kernel(x_src, x_dst, edge_src, edge_dst):
    N = x_src.shape[0]
    E = edge_src.shape[0]
    psub = -(-(-(-N // NUM_SUBCORES)) // 16) * 16
    nt = psub * NUM_SUBCORES
    xst = jnp.pad(x_src.T, ((0, 0), (0, nt - N)))
    xdt = jnp.pad(x_dst.T, ((0, 0), (0, nt - N)))
    vec_flat, _, _ = _sc_gather_make(N, E, chunk=1000)(
        xst, xdt, edge_src, edge_dst)
    vec_planes = vec_flat.reshape(3, E)
    sh_t, length, scal_t, cs, cn = _tc_math(vec_planes, blk=3200)
    return (sh_t.T, length, scal_t.T, cs, cn)


# speedup vs baseline: 1.3011x; 1.3011x over previous
import functools
import math

import jax
import jax.numpy as jnp
from jax import lax
from jax.experimental import pallas as pl
from jax.experimental.pallas import tpu as pltpu
from jax.experimental.pallas import tpu_sc as plsc

R = 3.0
LENGTH_ENC_DIM = 64
N_SIN = 10000
HALF_DIM = LENGTH_ENC_DIM // 2
EMB = math.log(N_SIN) / (HALF_DIM - 1)
SQ3 = math.sqrt(3.0)
SQ5 = math.sqrt(5.0)

SA, SB = 0.8 * R, 0.99 * R
NA, NB = 0.05 * R, 0.15 * R

NUM_CORES = 2
NUM_SUBCORES = 16
NW = NUM_CORES * NUM_SUBCORES

ROW_W = 16

TWO_PI_HI = 6.28125
TWO_PI_MID = 0.0019350051879882812
TWO_PI_LO = 3.0199159795074593e-07
INV_TWO_PI = 1.0 / (2.0 * math.pi)
SIN_C = (0.9999999528394287, -0.16666629708156558, 0.008332868412415909,
         -0.00019819996482129334, 2.711761652809948e-06,
         -2.082389059190004e-08)
COS_C = (0.9999999915888924, -0.49999991137407596, 0.0416665153020704,
         -0.0013887924190438086, 2.477237725272657e-05,
         -2.710259876049026e-07, 1.7328045475960078e-09)


def _poly_u(coeffs, u):
    r = coeffs[-1]
    for c in coeffs[-2::-1]:
        r = r * u + c
    return r


def _cos_0_pi(t):
    return _poly_u(COS_C, t * t)


def _sincos(ang):
    k = jnp.rint(ang * INV_TWO_PI)
    th = ang - k * TWO_PI_HI
    th = th - k * TWO_PI_MID
    th = th - k * TWO_PI_LO
    u = th * th
    return _poly_u(SIN_C, u) * th, _poly_u(COS_C, u)


def _sc_gather_make(N, E, chunk):
    assert E % NW == 0
    per_w = E // NW
    assert per_w % chunk == 0 and chunk % 8 == 0 and chunk % 4 == 0
    n_chunks = per_w // chunk
    mesh = plsc.VectorSubcoreMesh(core_axis_name="c", subcore_axis_name="s")

    psub = -(-N // NUM_SUBCORES)
    psub = -(-psub // 16) * 16
    nt = psub * NUM_SUBCORES

    @functools.partial(
        pl.kernel,
        out_type=[
            jax.ShapeDtypeStruct((3 * E,), jnp.float32),
            jax.ShapeDtypeStruct((NUM_CORES, nt, ROW_W), jnp.float32),
            jax.ShapeDtypeStruct((NUM_CORES, nt, ROW_W), jnp.float32),
        ],
        mesh=mesh,
        compiler_params=pltpu.CompilerParams(
            use_tc_tiling_on_sc=False, needs_layout_passes=False),
        scratch_types=[
            pltpu.VMEM((chunk,), jnp.int32),
            pltpu.VMEM((chunk,), jnp.int32),
            pltpu.VMEM((chunk, ROW_W), jnp.float32),
            pltpu.VMEM((chunk, ROW_W), jnp.float32),
            pltpu.VMEM((3 * chunk,), jnp.float32),
            pltpu.VMEM((3, psub), jnp.float32),
            pltpu.VMEM((psub, ROW_W), jnp.float32),
            pltpu.SemaphoreType.DMA,
            pltpu.SemaphoreType.DMA,
        ],
    )
    def sc_gather(xst_hbm, xdt_hbm, es_hbm, ed_hbm,
                  out_hbm, xs_tbl, xd_tbl,
                  idx_s, idx_d, rows_s, rows_d, diff, plane_v, tbl_v,
                  sem_s, sem_d):
        sid = lax.axis_index("s")
        core = lax.axis_index("c")
        wid = sid * NUM_CORES + core
        lane = lax.broadcasted_iota(jnp.int32, (16,), 0)
        rowoff = lane >> 2
        col = lane & 3
        spat = col * chunk + rowoff
        smask = col < 3

        n0 = sid * psub
        pcols = [lane - lane + p for p in range(3)]
        for xt, tbl in ((xst_hbm, xs_tbl), (xdt_hbm, xd_tbl)):
            for p in range(3):
                pltpu.sync_copy(xt.at[p, pl.ds(n0, psub)],
                                plane_v.at[p, pl.ds(0, psub)])

            def bbody(j, _):
                row = j * 16 + lane
                for p in range(3):
                    v = plane_v[p, pl.ds(j * 16, 16)]
                    plsc.store_scatter(tbl_v, [row, pcols[p]], v)
                return ()

            lax.fori_loop(0, psub // 16, bbody, (), unroll=4)
            pltpu.sync_copy(tbl_v, tbl.at[core, pl.ds(n0, psub)])
        plsc.subcore_barrier()
        xs_my = xs_tbl.at[core]
        xd_my = xd_tbl.at[core]

        for c in range(n_chunks):
            base = wid * per_w + c * chunk
            pltpu.sync_copy(es_hbm.at[pl.ds(base, chunk)], idx_s)
            pltpu.sync_copy(ed_hbm.at[pl.ds(base, chunk)], idx_d)
            cp_s = pltpu.async_copy(xs_my.at[idx_s], rows_s, sem_s)
            cp_d = pltpu.async_copy(xd_my.at[idx_d], rows_d, sem_d)
            cp_s.wait()
            cp_d.wait()

            def body(i, _):
                a = plsc.load_gather(rows_s, [i * 4 + rowoff, col])
                b = plsc.load_gather(rows_d, [i * 4 + rowoff, col])
                plsc.store_scatter(diff, [i * 4 + spat], a - b, mask=smask)
                return ()

            lax.fori_loop(0, chunk // 4, body, (), unroll=8)
            for p in range(3):
                pltpu.sync_copy(
                    diff.at[pl.ds(p * chunk, chunk)],
                    out_hbm.at[pl.ds(p * E + base, chunk)])

    return sc_gather


def _tc_math_body(E, blk, v_ref, sh_ref, len_ref, scal_ref, cs_ref, cn_ref):
    i = pl.program_id(0)
    off = pl.multiple_of(i * blk, blk)
    x = v_ref[pl.ds(off, blk)].reshape(1, blk)
    y = v_ref[pl.ds(E + off, blk)].reshape(1, blk)
    z = v_ref[pl.ds(2 * E + off, blk)].reshape(1, blk)
    l2 = x * x + y * y + z * z
    length = jnp.sqrt(l2 + 1e-12)
    inv = 1.0 / length
    ux = x * inv
    uy = y * inv
    uz = z * inv

    tf = jnp.clip((length - SA) / (SB - SA), 0.0, 1.0)
    fall = 0.5 * (1.0 + _cos_0_pi(jnp.pi * tf))
    tr = jnp.clip((length - NA) / (NB - NA), 0.0, 1.0)
    rise = 0.5 * (1.0 - _cos_0_pi(jnp.pi * tr))
    cs = fall
    cn = rise * fall
    cs_ref[pl.ds(off, blk)] = cs.reshape(blk)
    cn_ref[pl.ds(off, blk)] = cn.reshape(blk)
    len_ref[pl.ds(off, blk)] = length.reshape(blk)

    ii = lax.broadcasted_iota(jnp.int32, (HALF_DIM, 1), 0)
    freqs = jnp.exp(ii.astype(jnp.float32) * (-EMB))
    ang = (length * (N_SIN / R)) * freqs
    sin_a, cos_a = _sincos(ang)
    scal_ref[...] = jnp.concatenate([sin_a, cos_a], axis=0)

    sh1x = SQ3 * ux * cn
    sh1y = SQ3 * uy * cn
    sh1z = SQ3 * uz * cn
    c5 = SQ5 * cn
    sh2a = c5 * SQ3 * ux * uz
    sh2b = c5 * SQ3 * ux * uy
    sh2c = c5 * (uy * uy - 0.5 * (ux * ux + uz * uz))
    sh2d = c5 * SQ3 * uy * uz
    sh2e = c5 * (SQ3 / 2.0) * (uz * uz - ux * ux)
    sh_ref[...] = jnp.concatenate(
        [cs, sh1x, sh1y, sh1z, sh2a, sh2b, sh2c, sh2d, sh2e], axis=0)


def _tc_math(vec_flat, blk):
    E = vec_flat.shape[0] // 3
    assert E % blk == 0 and blk % 128 == 0
    grid = (E // blk,)
    out_shapes = [
        jax.ShapeDtypeStruct((9, E), jnp.float32),
        jax.ShapeDtypeStruct((E,), jnp.float32),
        jax.ShapeDtypeStruct((LENGTH_ENC_DIM, E), jnp.float32),
        jax.ShapeDtypeStruct((E,), jnp.float32),
        jax.ShapeDtypeStruct((E,), jnp.float32),
    ]
    out_specs = [
        pl.BlockSpec((9, blk), lambda i: (0, i)),
        pl.BlockSpec((E, ), lambda i: (0,)),
        pl.BlockSpec((LENGTH_ENC_DIM, blk), lambda i: (0, i)),
        pl.BlockSpec((E,), lambda i: (0,)),
        pl.BlockSpec((E,), lambda i: (0,)),
    ]
    return pl.pallas_call(
        functools.partial(_tc_math_body, E, blk),
        grid=grid,
        in_specs=[pl.BlockSpec((3 * E,), lambda i: (0,))],
        out_specs=out_specs,
        out_shape=out_shapes,
    )(vec_flat)


def kernel(x_src, x_dst, edge_src, edge_dst):
    N = x_src.shape[0]
    E = edge_src.shape[0]
    psub = -(-(-(-N // NUM_SUBCORES)) // 16) * 16
    nt = psub * NUM_SUBCORES
    xst = jnp.pad(x_src.T, ((0, 0), (0, nt - N)))
    xdt = jnp.pad(x_dst.T, ((0, 0), (0, nt - N)))
    vec_flat, _, _ = _sc_gather_make(N, E, chunk=1000)(
        xst, xdt, edge_src, edge_dst)
    sh_t, length, scal_t, cs, cn = _tc_math(vec_flat, blk=6400)
    return (sh_t.T, length, scal_t.T, cs, cn)
